# Initial kernel scaffold; baseline (speedup 1.0000x reference)
import jax
import jax.numpy as jnp
from jax.experimental import pallas as pl


def kernel(xyz, points, affine_alpha, affine_beta):
    raise NotImplementedError("write your pallas kernel here")


# speedup vs baseline: 3.6502x; 3.6502x over previous
import functools
import jax
import jax.numpy as jnp
from jax.experimental import pallas as pl
from jax.experimental.pallas import tpu as pltpu

_GROUPS = 512
_K = 32
_RADIUS = 0.2
_B = 16
_N = 2048
_C = 256
_STILE = 32


def _fps_body(x_ref, y_ref, z_ref, oh0_ref, cent_ref):
    x = x_ref[...]
    y = y_ref[...]
    z = z_ref[...]
    lane = jax.lax.broadcasted_iota(jnp.int32, (_B, _N), 1)
    s_lane = jax.lax.broadcasted_iota(jnp.int32, (_B, _GROUPS), 1)

    def body(i, carry):
        distance, oh, cent = carry
        cx = jnp.sum(x * oh, axis=1, keepdims=True)
        cy = jnp.sum(y * oh, axis=1, keepdims=True)
        cz = jnp.sum(z * oh, axis=1, keepdims=True)
        far = jnp.sum(jnp.where(oh > 0, lane, 0), axis=1, keepdims=True)
        cent = jnp.where(s_lane == i, far, cent)
        dx = x - cx
        dy = y - cy
        dz = z - cz
        d = dx * dx + dy * dy + dz * dz
        distance = jnp.minimum(distance, d)
        m = jnp.max(distance, axis=1, keepdims=True)
        nf = jnp.min(jnp.where(distance == m, lane, _N), axis=1, keepdims=True)
        oh = (lane == nf).astype(jnp.float32)
        return distance, oh, cent

    distance0 = jnp.full((_B, _N), 1e10, dtype=jnp.float32)
    cent0 = jnp.zeros((_B, _GROUPS), dtype=jnp.int32)
    _, _, cent = jax.lax.fori_loop(0, _GROUPS, body, (distance0, oh0_ref[...], cent0))
    cent_ref[...] = cent


def _ballq_body(x_ref, y_ref, z_ref, xyz_ref, fps_ref, idx_ref, nxyz_ref):
    xs = x_ref[0]
    ys = y_ref[0]
    zs = z_ref[0]
    cfar = fps_ref[0]
    lane = jax.lax.broadcasted_iota(jnp.int32, (_GROUPS, _N), 1)
    p = (lane == cfar).astype(jnp.float32)
    cx = jnp.sum(p * xs, axis=1, keepdims=True)
    cy = jnp.sum(p * ys, axis=1, keepdims=True)
    cz = jnp.sum(p * zs, axis=1, keepdims=True)
    nxyz = jnp.concatenate([cx, cy, cz], axis=1)
    nxyz_ref[0] = nxyz

    bcx = cx.astype(jnp.bfloat16).astype(jnp.float32)
    bcy = cy.astype(jnp.bfloat16).astype(jnp.float32)
    bcz = cz.astype(jnp.bfloat16).astype(jnp.float32)
    bxs = xs.astype(jnp.bfloat16).astype(jnp.float32)
    bys = ys.astype(jnp.bfloat16).astype(jnp.float32)
    bzs = zs.astype(jnp.bfloat16).astype(jnp.float32)
    mm = bcx * bxs + bcy * bys + bcz * bzs
    cnorm = cx * cx + cy * cy + cz * cz
    nnorm = xs * xs + ys * ys + zs * zs
    d = -2.0 * mm
    d = d + cnorm
    d = d + nnorm
    mask = jnp.logical_not(d > _RADIUS * _RADIUS)

    slots = []
    t = jnp.full((_GROUPS, 1), -1, dtype=jnp.int32)
    s0 = None
    for _ in range(_K):
        cand = jnp.where(jnp.logical_and(mask, lane > t), lane, _N)
        s = jnp.min(cand, axis=1, keepdims=True)
        if s0 is None:
            s0 = s
            slots.append(s)
        else:
            slots.append(jnp.where(s == _N, s0, s))
        t = s
    idx_ref[0] = jnp.concatenate(slots, axis=1)


def _gather_body(pts_ref, xyz_ref, idxf_ref, fps_ref, out_ref):
    idxf = idxf_ref[0]
    rows = _STILE * _K
    lane = jax.lax.broadcasted_iota(jnp.int32, (rows, _N), 1)
    g = (lane == idxf).astype(jnp.float32)
    pts = pts_ref[0]
    gp = jnp.dot(g, pts, preferred_element_type=jnp.float32,
                 precision=jax.lax.Precision.HIGHEST)
    gx = jnp.dot(g, xyz_ref[0], preferred_element_type=jnp.float32,
                 precision=jax.lax.Precision.HIGHEST)
    fpsb = fps_ref[0]
    a_lane = jax.lax.broadcasted_iota(jnp.int32, (_STILE, _N), 1)
    a = (a_lane == fpsb).astype(jnp.float32)
    ap = jnp.dot(a, pts, preferred_element_type=jnp.float32,
                 precision=jax.lax.Precision.HIGHEST)
    ap_rep = jnp.broadcast_to(ap[:, None, :], (_STILE, _K, _C)).reshape(rows, _C)
    out_ref[0] = jnp.concatenate([gp, gx, ap_rep], axis=1)


def kernel(xyz, points, affine_alpha, affine_beta):
    del affine_alpha, affine_beta
    B, N, _ = xyz.shape
    S = _GROUPS

    xT = jnp.transpose(xyz, (2, 0, 1))
    x, y, z = xT[0], xT[1], xT[2]

    far0 = jax.random.randint(jax.random.key(1), (B,), 0, N, dtype=jnp.int32)
    oh0 = (jnp.arange(N, dtype=jnp.int32)[None, :] == far0[:, None]).astype(jnp.float32)

    fps_idx = pl.pallas_call(
        _fps_body,
        out_shape=jax.ShapeDtypeStruct((B, S), jnp.int32),
    )(x, y, z, oh0)

    x3 = x.reshape(B, 1, N)
    y3 = y.reshape(B, 1, N)
    z3 = z.reshape(B, 1, N)
    fps_r = fps_idx.reshape(B, S, 1)

    idx, new_xyz = pl.pallas_call(
        _ballq_body,
        grid=(B,),
        in_specs=[
            pl.BlockSpec((1, 1, N), lambda b: (b, 0, 0)),
            pl.BlockSpec((1, 1, N), lambda b: (b, 0, 0)),
            pl.BlockSpec((1, 1, N), lambda b: (b, 0, 0)),
            pl.BlockSpec((1, N, 3), lambda b: (b, 0, 0)),
            pl.BlockSpec((1, S, 1), lambda b: (b, 0, 0)),
        ],
        out_specs=[
            pl.BlockSpec((1, S, _K), lambda b: (b, 0, 0)),
            pl.BlockSpec((1, S, 3), lambda b: (b, 0, 0)),
        ],
        out_shape=[
            jax.ShapeDtypeStruct((B, S, _K), jnp.int32),
            jax.ShapeDtypeStruct((B, S, 3), jnp.float32),
        ],
    )(x3, y3, z3, xyz, fps_r)

    idx_flat = idx.reshape(B, S * _K, 1)
    n_stiles = S // _STILE
    rows = _STILE * _K

    new_points = pl.pallas_call(
        _gather_body,
        grid=(B, n_stiles),
        in_specs=[
            pl.BlockSpec((1, N, _C), lambda b, s: (b, 0, 0)),
            pl.BlockSpec((1, N, 3), lambda b, s: (b, 0, 0)),
            pl.BlockSpec((1, rows, 1), lambda b, s: (b, s, 0)),
            pl.BlockSpec((1, _STILE, 1), lambda b, s: (b, s, 0)),
        ],
        out_specs=pl.BlockSpec((1, rows, 515), lambda b, s: (b, s, 0)),
        out_shape=jax.ShapeDtypeStruct((B, S * _K, 515), jnp.float32),
    )(points, xyz, idx_flat, fps_r)

    return new_xyz, new_points.reshape(B, S, _K, 515)
